# baseline (speedup 1.0000x reference)
import functools
import math

import jax
import jax.numpy as jnp
from jax import lax
from jax.experimental import pallas as pl
from jax.experimental.pallas import tpu as pltpu
from jax.experimental.pallas import tpu_sc as plsc

_VOCAB = 1000
_D = 128
_SCALE = math.sqrt(float(_D))

_NC = 2
_NS = 16
_NW = _NC * _NS

_CHUNK = 64
_NBUF = 8
_GPB = 1


_VPAD = 1024
_STG = _VPAD // _NS


def _make_sc_gather(batch, hist):
    assert batch % _NW == 0
    b_per_w = batch // _NW
    big = _GPB * _CHUNK
    assert b_per_w % big == 0
    ncb = b_per_w // big
    nchunks = hist * ncb
    assert nchunks % _NBUF == 0

    mesh = plsc.VectorSubcoreMesh(core_axis_name="c", subcore_axis_name="s")

    @functools.partial(
        pl.kernel,
        mesh=mesh,
        out_type=jax.ShapeDtypeStruct((hist, batch, _D), jnp.float32),
        scratch_types=[
            pltpu.VMEM((hist, b_per_w), jnp.int32),
            pltpu.VMEM((_NBUF, _GPB * _CHUNK, _D), jnp.float32),
            pltpu.VMEM((_STG, _D), jnp.float32),
            pltpu.VMEM_SHARED((_VPAD, _D), jnp.float32),
        ]
        + [pltpu.SemaphoreType.DMA] * (2 * _NBUF),
    )
    def k(table_hbm, idxT_hbm, out_hbm, idx_v, rows_v, stage_v, table_sh,
          *sems):
        gsems, ssems = sems[:_NBUF], sems[_NBUF:]
        sid = lax.axis_index("s")
        wid = sid * _NC + lax.axis_index("c")
        wb = wid * b_per_w
        tail = _VOCAB - (_NS - 1) * _STG

        @pl.when(sid < _NS - 1)
        def _():
            pltpu.sync_copy(table_hbm.at[pl.ds(sid * _STG, _STG)], stage_v)

        @pl.when(sid == _NS - 1)
        def _():
            pltpu.sync_copy(table_hbm.at[pl.ds((_NS - 1) * _STG, tail)],
                            stage_v.at[pl.ds(0, tail)])

        def scale_body(i, carry):
            r, kk = i // (_D // 16), i % (_D // 16)
            stage_v[r, pl.ds(kk * 16, 16)] = (
                stage_v[r, pl.ds(kk * 16, 16)] * _SCALE)
            return carry

        lax.fori_loop(0, _STG * (_D // 16), scale_body, 0)

        @pl.when(sid < _NS - 1)
        def _():
            pltpu.sync_copy(stage_v,
                            table_sh.at[pl.ds(sid * _STG, _STG)])

        @pl.when(sid == _NS - 1)
        def _():
            pltpu.sync_copy(stage_v.at[pl.ds(0, tail)],
                            table_sh.at[pl.ds((_NS - 1) * _STG, tail)])

        pltpu.sync_copy(idxT_hbm.at[:, pl.ds(wb, b_per_w)], idx_v)
        plsc.subcore_barrier()

        def gather(c, h, b):
            j, cb = c // ncb, c % ncb
            return pltpu.make_async_copy(
                table_sh.at[idx_v.at[
                    j, pl.ds(cb * big + h * _CHUNK, _CHUNK)]],
                rows_v.at[b, pl.ds(h * _CHUNK, _CHUNK)], gsems[b])

        def scatter(c, b):
            j, cb = c // ncb, c % ncb
            return pltpu.make_async_copy(
                rows_v.at[b],
                out_hbm.at[j, pl.ds(wb + cb * big, big)],
                ssems[b])

        for b in range(_NBUF):
            for h in range(_GPB):
                gather(b, h, b).start()

        def body(g, carry):
            for b in range(_NBUF):
                c = g * _NBUF + b
                for h in range(_GPB):
                    gather(c, h, b).wait()
                scatter(c, b).start()
            for b in range(_NBUF):
                c = g * _NBUF + b
                scatter(c, b).wait()
                for h in range(_GPB):
                    gather(c + _NBUF, h, b).start()
            return carry

        lax.fori_loop(0, nchunks // _NBUF - 1, body, 0)

        for b in range(_NBUF):
            c = nchunks - _NBUF + b
            for h in range(_GPB):
                gather(c, h, b).wait()
            scatter(c, b).start()
        for b in range(_NBUF):
            c = nchunks - _NBUF + b
            scatter(c, b).wait()

    return k


def kernel(x, W):
    batch, hist = x.shape
    outT = _make_sc_gather(batch, hist)(W, x.T)
    return jnp.transpose(outT, (1, 0, 2))


# speedup vs baseline: 1.0389x; 1.0389x over previous
import functools
import math

import jax
import jax.numpy as jnp
from jax import lax
from jax.experimental import pallas as pl
from jax.experimental.pallas import tpu as pltpu
from jax.experimental.pallas import tpu_sc as plsc

_VOCAB = 1000
_D = 128
_SCALE = math.sqrt(float(_D))

_NC = 2
_NS = 16
_NW = _NC * _NS

_CHUNK = 64
_NBUF = 10
_GPB = 1


_VPAD = 1024


def _scale_table_body(w_ref, o_ref):
    o_ref[pl.ds(0, _VOCAB), :] = w_ref[...] * _SCALE


def _scale_table(W):
    return pl.pallas_call(
        _scale_table_body,
        out_shape=jax.ShapeDtypeStruct((_VPAD, _D), W.dtype),
    )(W)


def _make_sc_gather(batch, hist):
    assert batch % _NW == 0
    b_per_w = batch // _NW
    big = _GPB * _CHUNK
    assert b_per_w % big == 0
    ncb = b_per_w // big
    nchunks = hist * ncb
    assert nchunks % _NBUF == 0

    mesh = plsc.VectorSubcoreMesh(core_axis_name="c", subcore_axis_name="s")

    @functools.partial(
        pl.kernel,
        mesh=mesh,
        out_type=jax.ShapeDtypeStruct((hist, batch, _D), jnp.float32),
        scratch_types=[
            pltpu.VMEM((hist, b_per_w), jnp.int32),
            pltpu.VMEM((_NBUF, _GPB * _CHUNK, _D), jnp.float32),
            pltpu.VMEM_SHARED((_VPAD, _D), jnp.float32),
        ]
        + [pltpu.SemaphoreType.DMA] * (2 * _NBUF),
    )
    def k(table_hbm, idxT_hbm, out_hbm, idx_v, rows_v, table_sh, *sems):
        gsems, ssems = sems[:_NBUF], sems[_NBUF:]
        sid = lax.axis_index("s")
        wid = sid * _NC + lax.axis_index("c")
        wb = wid * b_per_w
        stage = _VPAD // _NS
        pltpu.sync_copy(table_hbm.at[pl.ds(sid * stage, stage)],
                        table_sh.at[pl.ds(sid * stage, stage)])
        pltpu.sync_copy(idxT_hbm.at[:, pl.ds(wb, b_per_w)], idx_v)
        plsc.subcore_barrier()

        def gather(c, h, b):
            j, cb = c // ncb, c % ncb
            return pltpu.make_async_copy(
                table_sh.at[idx_v.at[
                    j, pl.ds(cb * big + h * _CHUNK, _CHUNK)]],
                rows_v.at[b, pl.ds(h * _CHUNK, _CHUNK)], gsems[b])

        def scatter(c, b):
            j, cb = c // ncb, c % ncb
            return pltpu.make_async_copy(
                rows_v.at[b],
                out_hbm.at[j, pl.ds(wb + cb * big, big)],
                ssems[b])

        for b in range(_NBUF):
            for h in range(_GPB):
                gather(b, h, b).start()

        def body(g, carry):
            for b in range(_NBUF):
                c = g * _NBUF + b
                for h in range(_GPB):
                    gather(c, h, b).wait()
                scatter(c, b).start()
            for b in range(_NBUF):
                c = g * _NBUF + b
                scatter(c, b).wait()
                for h in range(_GPB):
                    gather(c + _NBUF, h, b).start()
            return carry

        lax.fori_loop(0, nchunks // _NBUF - 1, body, 0)

        for b in range(_NBUF):
            c = nchunks - _NBUF + b
            for h in range(_GPB):
                gather(c, h, b).wait()
            scatter(c, b).start()
        for b in range(_NBUF):
            c = nchunks - _NBUF + b
            scatter(c, b).wait()

    return k


def kernel(x, W):
    batch, hist = x.shape
    Ws = _scale_table(W)
    outT = _make_sc_gather(batch, hist)(Ws, x.T)
    return jnp.transpose(outT, (1, 0, 2))
